# Initial kernel scaffold; baseline (speedup 1.0000x reference)
import functools

import jax
import jax.numpy as jnp
from jax import lax
from jax.experimental import pallas as pl
from jax.experimental.pallas import tpu as pltpu
from jax.experimental.pallas import tpu_sc as plsc

N = 10000
E = 320000
D = 128

NC = 2
NS = 16
NW = NC * NS
CHUNK = 128
CHUNKS = 80
E_TILE = CHUNK * CHUNKS
E_PAD = NW * E_TILE
ROWS_SUB = 626
N_PAD = NS * ROWS_SUB

_MESH = plsc.VectorSubcoreMesh(core_axis_name="core", subcore_axis_name="subcore")


@functools.partial(
    pl.kernel,
    out_type=jax.ShapeDtypeStruct((NC, N_PAD, 16), jnp.float32),
    mesh=_MESH,
    scratch_types=[
        pltpu.VMEM((CHUNKS, CHUNK), jnp.int32),
        pltpu.VMEM((CHUNK, 16), jnp.float32),
        pltpu.VMEM_SHARED((N_PAD, 16), jnp.float32),
    ],
)
def _sc_degree(dst_hbm, ones_hbm, zeros_hbm, out_hbm, dst_v, ones_v, acc):
    c = lax.axis_index("core")
    s = lax.axis_index("subcore")
    wid = c * NS + s
    pltpu.sync_copy(zeros_hbm, acc.at[pl.ds(s * ROWS_SUB, ROWS_SUB)])
    pltpu.sync_copy(ones_hbm, ones_v)
    pltpu.sync_copy(dst_hbm.at[wid], dst_v)
    plsc.subcore_barrier()

    @pl.loop(0, CHUNKS)
    def _(j):
        pltpu.sync_copy(ones_v, acc.at[dst_v.at[j]], add=True)

    plsc.subcore_barrier()
    pltpu.sync_copy(acc.at[pl.ds(s * ROWS_SUB, ROWS_SUB)],
                    out_hbm.at[c, pl.ds(s * ROWS_SUB, ROWS_SUB)])


@functools.partial(
    pl.kernel,
    out_type=jax.ShapeDtypeStruct((NC, N_PAD, D), jnp.float32),
    mesh=_MESH,
    scratch_types=[
        pltpu.VMEM((CHUNKS, CHUNK), jnp.int32),
        pltpu.VMEM((CHUNKS, CHUNK), jnp.int32),
        pltpu.VMEM((CHUNK, D), jnp.float32),
        pltpu.VMEM_SHARED((N_PAD, D), jnp.float32),
    ],
)
def _sc_aggregate(g_hbm, src_hbm, dst_hbm, zeros_hbm, out_hbm,
                  src_v, dst_v, rows_v, acc):
    c = lax.axis_index("core")
    s = lax.axis_index("subcore")
    wid = c * NS + s
    pltpu.sync_copy(zeros_hbm, acc.at[pl.ds(s * ROWS_SUB, ROWS_SUB)])
    pltpu.sync_copy(src_hbm.at[wid], src_v)
    pltpu.sync_copy(dst_hbm.at[wid], dst_v)
    plsc.subcore_barrier()

    @pl.loop(0, CHUNKS)
    def _(j):
        pltpu.sync_copy(g_hbm.at[src_v.at[j]], rows_v)
        pltpu.sync_copy(rows_v, acc.at[dst_v.at[j]], add=True)

    plsc.subcore_barrier()
    pltpu.sync_copy(acc.at[pl.ds(s * ROWS_SUB, ROWS_SUB)],
                    out_hbm.at[c, pl.ds(s * ROWS_SUB, ROWS_SUB)])


def _dinv_from_counts(cnt_ref):
    deg = 1.0 + (cnt_ref[0] + cnt_ref[1])[:, 0:1]
    return lax.rsqrt(deg)


def _tc_first(cnt_ref, x_ref, w_ref, g_ref):
    dinv = _dinv_from_counts(cnt_ref)
    h = jnp.dot(x_ref[...], w_ref[...], preferred_element_type=jnp.float32)
    g_ref[...] = dinv * h


def _tc_mid(cnt_ref, a_ref, g_ref, b_ref, w_ref, g2_ref):
    dinv = _dinv_from_counts(cnt_ref)
    z = dinv * (a_ref[0] + a_ref[1] + g_ref[...]) + b_ref[...]
    z = jnp.maximum(z, 0.0)
    h = jnp.dot(z, w_ref[...], preferred_element_type=jnp.float32)
    g2_ref[...] = dinv * h


def _tc_last(cnt_ref, a_ref, g_ref, b_ref, out_ref):
    dinv = _dinv_from_counts(cnt_ref)
    out_ref[...] = dinv * (a_ref[0] + a_ref[1] + g_ref[...]) + b_ref[...]


def _call_tc(body, *args):
    return pl.pallas_call(
        body,
        out_shape=jax.ShapeDtypeStruct((N_PAD, D), jnp.float32),
    )(*args)


def kernel(x, edge_index, W1, b1, W2, b2):
    src = edge_index[0].astype(jnp.int32)
    dst = edge_index[1].astype(jnp.int32)
    pad = jnp.full((E_PAD - E,), N, dtype=jnp.int32)
    src3 = jnp.concatenate([src, pad]).reshape(NW, CHUNKS, CHUNK)
    dst3 = jnp.concatenate([dst, pad]).reshape(NW, CHUNKS, CHUNK)
    x_pad = jnp.pad(x, ((0, N_PAD - N), (0, 0)))

    ones16 = jnp.ones((CHUNK, 16), jnp.float32)
    zeros16 = jnp.zeros((ROWS_SUB, 16), jnp.float32)
    zerosD = jnp.zeros((ROWS_SUB, D), jnp.float32)
    b1r = b1.reshape(1, D)
    b2r = b2.reshape(1, D)

    cnt = _sc_degree(dst3, ones16, zeros16)
    g1 = _call_tc(_tc_first, cnt, x_pad, W1)
    a1 = _sc_aggregate(g1, src3, dst3, zerosD)
    g2 = _call_tc(_tc_mid, cnt, a1, g1, b1r, W2)
    a2 = _sc_aggregate(g2, src3, dst3, zerosD)
    out = _call_tc(_tc_last, cnt, a2, g2, b2r)
    return out[:N]


# speedup vs baseline: 7.7702x; 7.7702x over previous
import functools

import jax
import jax.numpy as jnp
from jax import lax
from jax.experimental import pallas as pl
from jax.experimental.pallas import tpu as pltpu
from jax.experimental.pallas import tpu_sc as plsc

N = 10000
E = 320000
D = 128

NC = 2
NS = 16
NW = NC * NS
CHUNK = 128
CHUNKS = 80
E_TILE = CHUNK * CHUNKS
E_PAD = NW * E_TILE
ROWS_SUB = 632
N_PAD = NS * ROWS_SUB

_MESH = plsc.VectorSubcoreMesh(core_axis_name="core", subcore_axis_name="subcore")


@functools.partial(
    pl.kernel,
    out_type=jax.ShapeDtypeStruct((NC, N_PAD, D), jnp.float32),
    mesh=_MESH,
    scratch_types=[
        pltpu.VMEM((CHUNKS, CHUNK), jnp.int32),
        pltpu.VMEM((CHUNK, D), jnp.float32),
        pltpu.VMEM_SHARED((N_PAD, D), jnp.float32),
    ],
)
def _sc_degree(dst_hbm, ones_hbm, zeros_hbm, out_hbm, dst_v, ones_v, acc):
    c = lax.axis_index("core")
    s = lax.axis_index("subcore")
    wid = c * NS + s
    pltpu.sync_copy(zeros_hbm, acc.at[pl.ds(s * ROWS_SUB, ROWS_SUB)])
    pltpu.sync_copy(ones_hbm, ones_v)
    pltpu.sync_copy(dst_hbm.at[wid], dst_v)
    plsc.subcore_barrier()

    @pl.loop(0, CHUNKS)
    def _(j):
        pltpu.sync_copy(ones_v, acc.at[dst_v.at[j]], add=True)

    plsc.subcore_barrier()
    pltpu.sync_copy(acc.at[pl.ds(s * ROWS_SUB, ROWS_SUB)],
                    out_hbm.at[c, pl.ds(s * ROWS_SUB, ROWS_SUB)])


@functools.partial(
    pl.kernel,
    out_type=jax.ShapeDtypeStruct((NC, N_PAD, D), jnp.float32),
    mesh=_MESH,
    scratch_types=[
        pltpu.VMEM((CHUNKS, CHUNK), jnp.int32),
        pltpu.VMEM((CHUNKS, CHUNK), jnp.int32),
        pltpu.VMEM((CHUNK, D), jnp.float32),
        pltpu.VMEM_SHARED((N_PAD, D), jnp.float32),
    ],
)
def _sc_aggregate(g_hbm, src_hbm, dst_hbm, zeros_hbm, out_hbm,
                  src_v, dst_v, rows_v, acc):
    c = lax.axis_index("core")
    s = lax.axis_index("subcore")
    wid = c * NS + s
    pltpu.sync_copy(zeros_hbm, acc.at[pl.ds(s * ROWS_SUB, ROWS_SUB)])
    pltpu.sync_copy(src_hbm.at[wid], src_v)
    pltpu.sync_copy(dst_hbm.at[wid], dst_v)
    plsc.subcore_barrier()

    @pl.loop(0, CHUNKS)
    def _(j):
        pltpu.sync_copy(g_hbm.at[src_v.at[j]], rows_v)
        pltpu.sync_copy(rows_v, acc.at[dst_v.at[j]], add=True)

    plsc.subcore_barrier()
    pltpu.sync_copy(acc.at[pl.ds(s * ROWS_SUB, ROWS_SUB)],
                    out_hbm.at[c, pl.ds(s * ROWS_SUB, ROWS_SUB)])


def _dinv_from_counts(cnt_ref):
    deg = 1.0 + (cnt_ref[0] + cnt_ref[1])[:, 0:1]
    return lax.rsqrt(deg)


def _tc_first(cnt_ref, x_ref, w_ref, g_ref):
    dinv = _dinv_from_counts(cnt_ref)
    h = jnp.dot(x_ref[...], w_ref[...], preferred_element_type=jnp.float32)
    g_ref[...] = dinv * h


def _tc_mid(cnt_ref, a_ref, g_ref, b_ref, w_ref, g2_ref):
    dinv = _dinv_from_counts(cnt_ref)
    z = dinv * (a_ref[0] + a_ref[1] + g_ref[...]) + b_ref[...]
    z = jnp.maximum(z, 0.0)
    h = jnp.dot(z, w_ref[...], preferred_element_type=jnp.float32)
    g2_ref[...] = dinv * h


def _tc_last(cnt_ref, a_ref, g_ref, b_ref, out_ref):
    dinv = _dinv_from_counts(cnt_ref)
    out_ref[...] = dinv * (a_ref[0] + a_ref[1] + g_ref[...]) + b_ref[...]


def _call_tc(body, *args):
    return pl.pallas_call(
        body,
        out_shape=jax.ShapeDtypeStruct((N_PAD, D), jnp.float32),
    )(*args)


def kernel(x, edge_index, W1, b1, W2, b2):
    src = edge_index[0].astype(jnp.int32)
    dst = edge_index[1].astype(jnp.int32)
    pad = jnp.full((E_PAD - E,), N, dtype=jnp.int32)
    src3 = jnp.concatenate([src, pad]).reshape(NW, CHUNKS, CHUNK)
    dst3 = jnp.concatenate([dst, pad]).reshape(NW, CHUNKS, CHUNK)
    x_pad = jnp.pad(x, ((0, N_PAD - N), (0, 0)))

    onesD = jnp.ones((CHUNK, D), jnp.float32)
    zerosD = jnp.zeros((ROWS_SUB, D), jnp.float32)
    b1r = b1.reshape(1, D)
    b2r = b2.reshape(1, D)

    cnt = _sc_degree(dst3, onesD, zerosD)
    g1 = _call_tc(_tc_first, cnt, x_pad, W1)
    a1 = _sc_aggregate(g1, src3, dst3, zerosD)
    g2 = _call_tc(_tc_mid, cnt, a1, g1, b1r, W2)
    a2 = _sc_aggregate(g2, src3, dst3, zerosD)
    out = _call_tc(_tc_last, cnt, a2, g2, b2r)
    return out[:N]


# speedup vs baseline: 8.4542x; 1.0880x over previous
import functools

import jax
import jax.numpy as jnp
from jax import lax
from jax.experimental import pallas as pl
from jax.experimental.pallas import tpu as pltpu
from jax.experimental.pallas import tpu_sc as plsc

N = 10000
E = 320000
D = 128

NC = 2
NS = 16
NW = NC * NS
CHUNK = 128
CHUNKS = 80
E_TILE = CHUNK * CHUNKS
E_PAD = NW * E_TILE
ROWS_SUB = 632
N_PAD = NS * ROWS_SUB
NBUF = 2
H = 2
CH = CHUNKS // H
DEG_RING = 4

_MESH = plsc.VectorSubcoreMesh(core_axis_name="core", subcore_axis_name="subcore")


@functools.partial(
    pl.kernel,
    out_type=jax.ShapeDtypeStruct((NC, N_PAD, D), jnp.float32),
    mesh=_MESH,
    scratch_types=[
        pltpu.VMEM((CHUNKS, CHUNK), jnp.int32),
        pltpu.VMEM((CHUNK, D), jnp.float32),
        pltpu.VMEM_SHARED((N_PAD, D), jnp.float32),
    ] + [pltpu.SemaphoreType.DMA] * DEG_RING,
)
def _sc_degree(dst_hbm, ones_hbm, zeros_hbm, out_hbm, dst_v, ones_v, acc,
               *sems):
    c = lax.axis_index("core")
    s = lax.axis_index("subcore")
    wid = c * NS + s
    pltpu.sync_copy(zeros_hbm, acc.at[pl.ds(s * ROWS_SUB, ROWS_SUB)])
    pltpu.sync_copy(ones_hbm, ones_v)
    pltpu.sync_copy(dst_hbm.at[wid], dst_v)
    plsc.subcore_barrier()

    @pl.loop(0, CHUNKS, step=DEG_RING)
    def _(jo):
        for b in range(DEG_RING):
            j = jo + b

            @pl.when(jo > 0)
            def _():
                pltpu.make_async_copy(ones_v, acc.at[dst_v.at[j]],
                                      sems[b]).wait()

            pltpu.async_copy(ones_v, acc.at[dst_v.at[j]], sems[b], add=True)

    for b in range(DEG_RING):
        pltpu.make_async_copy(ones_v, acc.at[dst_v.at[b]], sems[b]).wait()

    plsc.subcore_barrier()
    pltpu.sync_copy(acc.at[pl.ds(s * ROWS_SUB, ROWS_SUB)],
                    out_hbm.at[c, pl.ds(s * ROWS_SUB, ROWS_SUB)])


@functools.partial(
    pl.kernel,
    out_type=jax.ShapeDtypeStruct((NC, N_PAD, D), jnp.float32),
    mesh=_MESH,
    scratch_types=[
        pltpu.VMEM((CH, CHUNK), jnp.int32),
        pltpu.VMEM((CH, CHUNK), jnp.int32),
        pltpu.VMEM((NBUF, CHUNK, D), jnp.float32),
        pltpu.VMEM_SHARED((N_PAD, D), jnp.float32),
    ] + [pltpu.SemaphoreType.DMA] * (2 * NBUF),
)
def _sc_aggregate(g_hbm, src_hbm, dst_hbm, zeros_hbm, out_hbm,
                  src_v, dst_v, rows_v, acc, *sems):
    gsem = sems[:NBUF]
    ssem = sems[NBUF:]
    c = lax.axis_index("core")
    s = lax.axis_index("subcore")
    wid = c * NS + s
    pltpu.sync_copy(zeros_hbm, acc.at[pl.ds(s * ROWS_SUB, ROWS_SUB)])
    plsc.subcore_barrier()

    for h in range(H):
        pltpu.sync_copy(src_hbm.at[wid, pl.ds(h * CH, CH)], src_v)
        pltpu.sync_copy(dst_hbm.at[wid, pl.ds(h * CH, CH)], dst_v)
        pltpu.async_copy(g_hbm.at[src_v.at[0]], rows_v.at[0], gsem[0])

        @pl.loop(0, CH, step=NBUF)
        def _(jo):
            for b in range(NBUF):
                j = jo + b
                b1 = 1 - b

                @pl.when(j >= 1)
                def _():
                    pltpu.make_async_copy(rows_v.at[b1], acc.at[dst_v.at[j]],
                                          ssem[b1]).wait()

                jg = jnp.where(j + 1 >= CH, 0, j + 1)
                pltpu.async_copy(g_hbm.at[src_v.at[jg]], rows_v.at[b1],
                                 gsem[b1])
                pltpu.make_async_copy(g_hbm.at[src_v.at[j]], rows_v.at[b],
                                      gsem[b]).wait()
                pltpu.async_copy(rows_v.at[b], acc.at[dst_v.at[j]], ssem[b],
                                 add=True)

        pltpu.make_async_copy(rows_v.at[1], acc.at[dst_v.at[0]],
                              ssem[1]).wait()
        pltpu.make_async_copy(g_hbm.at[src_v.at[0]], rows_v.at[0],
                              gsem[0]).wait()

    plsc.subcore_barrier()
    pltpu.sync_copy(acc.at[pl.ds(s * ROWS_SUB, ROWS_SUB)],
                    out_hbm.at[c, pl.ds(s * ROWS_SUB, ROWS_SUB)])


def _dinv_from_counts(cnt_ref):
    deg = 1.0 + (cnt_ref[0] + cnt_ref[1])[:, 0:1]
    return lax.rsqrt(deg)


def _tc_first(cnt_ref, x_ref, w_ref, g_ref):
    dinv = _dinv_from_counts(cnt_ref)
    h = jnp.dot(x_ref[...], w_ref[...], preferred_element_type=jnp.float32)
    g_ref[...] = dinv * h


def _tc_mid(cnt_ref, a_ref, g_ref, b_ref, w_ref, g2_ref):
    dinv = _dinv_from_counts(cnt_ref)
    z = dinv * (a_ref[0] + a_ref[1] + g_ref[...]) + b_ref[...]
    z = jnp.maximum(z, 0.0)
    h = jnp.dot(z, w_ref[...], preferred_element_type=jnp.float32)
    g2_ref[...] = dinv * h


def _tc_last(cnt_ref, a_ref, g_ref, b_ref, out_ref):
    dinv = _dinv_from_counts(cnt_ref)
    out_ref[...] = dinv * (a_ref[0] + a_ref[1] + g_ref[...]) + b_ref[...]


def _call_tc(body, *args):
    return pl.pallas_call(
        body,
        out_shape=jax.ShapeDtypeStruct((N_PAD, D), jnp.float32),
    )(*args)


def kernel(x, edge_index, W1, b1, W2, b2):
    src = edge_index[0].astype(jnp.int32)
    dst = edge_index[1].astype(jnp.int32)
    pad = jnp.full((E_PAD - E,), N, dtype=jnp.int32)
    src3 = jnp.concatenate([src, pad]).reshape(NW, CHUNKS, CHUNK)
    dst3 = jnp.concatenate([dst, pad]).reshape(NW, CHUNKS, CHUNK)
    x_pad = jnp.pad(x, ((0, N_PAD - N), (0, 0)))

    onesD = jnp.ones((CHUNK, D), jnp.float32)
    zerosD = jnp.zeros((ROWS_SUB, D), jnp.float32)
    b1r = b1.reshape(1, D)
    b2r = b2.reshape(1, D)

    cnt = _sc_degree(dst3, onesD, zerosD)
    g1 = _call_tc(_tc_first, cnt, x_pad, W1)
    a1 = _sc_aggregate(g1, src3, dst3, zerosD)
    g2 = _call_tc(_tc_mid, cnt, a1, g1, b1r, W2)
    a2 = _sc_aggregate(g2, src3, dst3, zerosD)
    out = _call_tc(_tc_last, cnt, a2, g2, b2r)
    return out[:N]


# speedup vs baseline: 24.6441x; 2.9150x over previous
import functools

import jax
import jax.numpy as jnp
from jax import lax
from jax.experimental import pallas as pl
from jax.experimental.pallas import tpu as pltpu
from jax.experimental.pallas import tpu_sc as plsc

N = 10000
E = 320000
D = 128

NC = 2
NS = 16
NW = NC * NS
CHUNK = 128
CHUNKS = 80
E_TILE = CHUNK * CHUNKS
E_PAD = NW * E_TILE
ROWS_SUB = 632
N_PAD = NS * ROWS_SUB
NBUF = 2
H = 2
CH = CHUNKS // H
DEG_RING = 4

_MESH = plsc.VectorSubcoreMesh(core_axis_name="core", subcore_axis_name="subcore")


@functools.partial(
    pl.kernel,
    out_type=jax.ShapeDtypeStruct((NC, N_PAD, D), jnp.float32),
    mesh=_MESH,
    scratch_types=[
        pltpu.VMEM((CHUNKS, CHUNK), jnp.int32),
        pltpu.VMEM((CHUNK, D), jnp.float32),
        pltpu.VMEM_SHARED((N_PAD, D), jnp.float32),
    ] + [pltpu.SemaphoreType.DMA] * DEG_RING,
)
def _sc_degree(dst_hbm, ones_hbm, zeros_hbm, out_hbm, dst_v, ones_v, acc,
               *sems):
    c = lax.axis_index("core")
    s = lax.axis_index("subcore")
    wid = c * NS + s
    pltpu.sync_copy(zeros_hbm, acc.at[pl.ds(s * ROWS_SUB, ROWS_SUB)])
    pltpu.sync_copy(ones_hbm, ones_v)
    pltpu.sync_copy(dst_hbm.at[wid], dst_v)
    plsc.subcore_barrier()

    @pl.loop(0, CHUNKS, step=DEG_RING)
    def _(jo):
        for b in range(DEG_RING):
            j = jo + b

            @pl.when(jo > 0)
            def _():
                pltpu.make_async_copy(ones_v, acc.at[dst_v.at[j]],
                                      sems[b]).wait()

            pltpu.async_copy(ones_v, acc.at[dst_v.at[j]], sems[b], add=True)

    for b in range(DEG_RING):
        pltpu.make_async_copy(ones_v, acc.at[dst_v.at[b]], sems[b]).wait()

    plsc.subcore_barrier()
    pltpu.sync_copy(acc.at[pl.ds(s * ROWS_SUB, ROWS_SUB)],
                    out_hbm.at[c, pl.ds(s * ROWS_SUB, ROWS_SUB)])


@functools.partial(
    pl.kernel,
    out_type=jax.ShapeDtypeStruct((NC, N_PAD, D), jnp.float32),
    mesh=_MESH,
    scratch_types=[
        pltpu.VMEM((CH, CHUNK), jnp.int32),
        pltpu.VMEM((CH, CHUNK), jnp.int32),
        pltpu.VMEM((NBUF, CHUNK, D), jnp.float32),
        pltpu.VMEM_SHARED((N_PAD, D), jnp.float32),
    ] + [pltpu.SemaphoreType.DMA] * (2 * NBUF),
)
def _sc_aggregate(g_hbm, src_hbm, dst_hbm, zeros_hbm, out_hbm,
                  src_v, dst_v, rows_v, acc, *sems):
    gsem = sems[:NBUF]
    ssem = sems[NBUF:]
    c = lax.axis_index("core")
    s = lax.axis_index("subcore")
    wid = c * NS + s
    pltpu.sync_copy(zeros_hbm, acc.at[pl.ds(s * ROWS_SUB, ROWS_SUB)])
    plsc.subcore_barrier()

    for h in range(H):
        pltpu.sync_copy(src_hbm.at[wid, pl.ds(h * CH, CH)], src_v)
        pltpu.sync_copy(dst_hbm.at[wid, pl.ds(h * CH, CH)], dst_v)
        pltpu.async_copy(g_hbm.at[src_v.at[0]], rows_v.at[0], gsem[0])

        @pl.loop(0, CH, step=NBUF)
        def _(jo):
            for b in range(NBUF):
                j = jo + b
                b1 = 1 - b

                @pl.when(j >= 1)
                def _():
                    pltpu.make_async_copy(rows_v.at[b1], acc.at[dst_v.at[j]],
                                          ssem[b1]).wait()

                jg = jnp.where(j + 1 >= CH, 0, j + 1)
                pltpu.async_copy(g_hbm.at[src_v.at[jg]], rows_v.at[b1],
                                 gsem[b1])
                pltpu.make_async_copy(g_hbm.at[src_v.at[j]], rows_v.at[b],
                                      gsem[b]).wait()
                pltpu.async_copy(rows_v.at[b], acc.at[dst_v.at[j]], ssem[b],
                                 add=True)

        pltpu.make_async_copy(rows_v.at[1], acc.at[dst_v.at[0]],
                              ssem[1]).wait()
        pltpu.make_async_copy(g_hbm.at[src_v.at[0]], rows_v.at[0],
                              gsem[0]).wait()

    plsc.subcore_barrier()
    pltpu.sync_copy(acc.at[pl.ds(s * ROWS_SUB, ROWS_SUB)],
                    out_hbm.at[c, pl.ds(s * ROWS_SUB, ROWS_SUB)])


def _dinv_from_counts(cnt_ref):
    deg = 1.0 + (cnt_ref[0] + cnt_ref[1])[:, 0:1]
    return lax.rsqrt(deg)


def _tc_first(cnt_ref, x_ref, w_ref, g_ref):
    dinv = _dinv_from_counts(cnt_ref)
    h = jnp.dot(x_ref[...], w_ref[...], preferred_element_type=jnp.float32)
    g_ref[...] = dinv * h


def _tc_mid(cnt_ref, a_ref, g_ref, b_ref, w_ref, g2_ref):
    dinv = _dinv_from_counts(cnt_ref)
    z = dinv * (a_ref[0] + a_ref[1] + g_ref[...]) + b_ref[...]
    z = jnp.maximum(z, 0.0)
    h = jnp.dot(z, w_ref[...], preferred_element_type=jnp.float32)
    g2_ref[...] = dinv * h


def _tc_last(cnt_ref, a_ref, g_ref, b_ref, out_ref):
    dinv = _dinv_from_counts(cnt_ref)
    out_ref[...] = dinv * (a_ref[0] + a_ref[1] + g_ref[...]) + b_ref[...]


def _call_tc(body, *args):
    return pl.pallas_call(
        body,
        out_shape=jax.ShapeDtypeStruct((N_PAD, D), jnp.float32),
    )(*args)


def kernel(x, edge_index, W1, b1, W2, b2):
    src = edge_index[0].astype(jnp.int32)
    dst = edge_index[1].astype(jnp.int32)
    n_dummy = E_PAD // NW - E // NW
    dummy = N + (jnp.arange(n_dummy, dtype=jnp.int32) % (N_PAD - N))
    dummy = jnp.broadcast_to(dummy, (NW, n_dummy))
    src3 = jnp.concatenate([src.reshape(NW, E // NW), dummy],
                           axis=1).reshape(NW, CHUNKS, CHUNK)
    dst3 = jnp.concatenate([dst.reshape(NW, E // NW), dummy],
                           axis=1).reshape(NW, CHUNKS, CHUNK)
    x_pad = jnp.pad(x, ((0, N_PAD - N), (0, 0)))

    onesD = jnp.ones((CHUNK, D), jnp.float32)
    zerosD = jnp.zeros((ROWS_SUB, D), jnp.float32)
    b1r = b1.reshape(1, D)
    b2r = b2.reshape(1, D)

    cnt = _sc_degree(dst3, onesD, zerosD)
    g1 = _call_tc(_tc_first, cnt, x_pad, W1)
    a1 = _sc_aggregate(g1, src3, dst3, zerosD)
    g2 = _call_tc(_tc_mid, cnt, a1, g1, b1r, W2)
    a2 = _sc_aggregate(g2, src3, dst3, zerosD)
    out = _call_tc(_tc_last, cnt, a2, g2, b2r)
    return out[:N]


# speedup vs baseline: 27.4753x; 1.1149x over previous
import functools

import jax
import jax.numpy as jnp
from jax import lax
from jax.experimental import pallas as pl
from jax.experimental.pallas import tpu as pltpu
from jax.experimental.pallas import tpu_sc as plsc

N = 10000
E = 320000
D = 128

NC = 2
NS = 16
NW = NC * NS
CHUNK = 128
CHUNKS = 80
E_TILE = CHUNK * CHUNKS
E_PAD = NW * E_TILE
ROWS_SUB = 640
N_PAD = NS * ROWS_SUB
NBUF = 2
H = 2
CH = CHUNKS // H
DEG_RING = 4

_MESH = plsc.VectorSubcoreMesh(core_axis_name="core", subcore_axis_name="subcore")


@functools.partial(
    pl.kernel,
    out_type=jax.ShapeDtypeStruct((NC, N_PAD, D), jnp.float32),
    mesh=_MESH,
    scratch_types=[
        pltpu.VMEM((CHUNKS, CHUNK), jnp.int32),
        pltpu.VMEM((CHUNK, D), jnp.float32),
        pltpu.VMEM_SHARED((N_PAD, D), jnp.float32),
    ] + [pltpu.SemaphoreType.DMA] * DEG_RING,
)
def _sc_degree(dst_hbm, ones_hbm, zeros_hbm, out_hbm, dst_v, ones_v, acc,
               *sems):
    c = lax.axis_index("core")
    s = lax.axis_index("subcore")
    wid = c * NS + s
    pltpu.sync_copy(zeros_hbm, acc.at[pl.ds(s * ROWS_SUB, ROWS_SUB)])
    pltpu.sync_copy(ones_hbm, ones_v)
    pltpu.sync_copy(dst_hbm.at[wid], dst_v)
    plsc.subcore_barrier()

    @pl.loop(0, CHUNKS, step=DEG_RING)
    def _(jo):
        for b in range(DEG_RING):
            j = jo + b

            @pl.when(jo > 0)
            def _():
                pltpu.make_async_copy(ones_v, acc.at[dst_v.at[j]],
                                      sems[b]).wait()

            pltpu.async_copy(ones_v, acc.at[dst_v.at[j]], sems[b], add=True)

    for b in range(DEG_RING):
        pltpu.make_async_copy(ones_v, acc.at[dst_v.at[b]], sems[b]).wait()

    plsc.subcore_barrier()
    pltpu.sync_copy(acc.at[pl.ds(s * ROWS_SUB, ROWS_SUB)],
                    out_hbm.at[c, pl.ds(s * ROWS_SUB, ROWS_SUB)])


@functools.partial(
    pl.kernel,
    out_type=jax.ShapeDtypeStruct((NC, N_PAD, D), jnp.float32),
    mesh=_MESH,
    scratch_types=[
        pltpu.VMEM((CH, CHUNK), jnp.int32),
        pltpu.VMEM((CH, CHUNK), jnp.int32),
        pltpu.VMEM((NBUF, CHUNK, D), jnp.float32),
        pltpu.VMEM_SHARED((N_PAD, D), jnp.float32),
    ] + [pltpu.SemaphoreType.DMA] * (2 * NBUF),
)
def _sc_aggregate(g_hbm, src_hbm, dst_hbm, zeros_hbm, out_hbm,
                  src_v, dst_v, rows_v, acc, *sems):
    gsem = sems[:NBUF]
    ssem = sems[NBUF:]
    c = lax.axis_index("core")
    s = lax.axis_index("subcore")
    wid = c * NS + s
    pltpu.sync_copy(zeros_hbm, acc.at[pl.ds(s * ROWS_SUB, ROWS_SUB)])
    plsc.subcore_barrier()

    for h in range(H):
        pltpu.sync_copy(src_hbm.at[wid, pl.ds(h * CH, CH)], src_v)
        pltpu.sync_copy(dst_hbm.at[wid, pl.ds(h * CH, CH)], dst_v)
        pltpu.async_copy(g_hbm.at[src_v.at[0]], rows_v.at[0], gsem[0])

        @pl.loop(0, CH, step=NBUF)
        def _(jo):
            for b in range(NBUF):
                j = jo + b
                b1 = 1 - b

                @pl.when(j >= 1)
                def _():
                    pltpu.make_async_copy(rows_v.at[b1], acc.at[dst_v.at[j]],
                                          ssem[b1]).wait()

                jg = jnp.where(j + 1 >= CH, 0, j + 1)
                pltpu.async_copy(g_hbm.at[src_v.at[jg]], rows_v.at[b1],
                                 gsem[b1])
                pltpu.make_async_copy(g_hbm.at[src_v.at[j]], rows_v.at[b],
                                      gsem[b]).wait()
                pltpu.async_copy(rows_v.at[b], acc.at[dst_v.at[j]], ssem[b],
                                 add=True)

        pltpu.make_async_copy(rows_v.at[1], acc.at[dst_v.at[0]],
                              ssem[1]).wait()
        pltpu.make_async_copy(g_hbm.at[src_v.at[0]], rows_v.at[0],
                              gsem[0]).wait()

    plsc.subcore_barrier()
    pltpu.sync_copy(acc.at[pl.ds(s * ROWS_SUB, ROWS_SUB)],
                    out_hbm.at[c, pl.ds(s * ROWS_SUB, ROWS_SUB)])


def _dinv_from_counts(cnt_ref):
    s = (cnt_ref[0] + cnt_ref[1])[:, 0:1].astype(jnp.float32)
    return lax.rsqrt(1.0 + s)


def _tc_first(cnt_ref, x_ref, w_ref, g_ref):
    dinv = _dinv_from_counts(cnt_ref)
    h = jnp.dot(x_ref[...], w_ref[...], preferred_element_type=jnp.float32)
    g_ref[...] = dinv * h


def _tc_mid(cnt_ref, a_ref, g_ref, b_ref, w_ref, g2_ref):
    dinv = _dinv_from_counts(cnt_ref)
    z = dinv * (a_ref[0] + a_ref[1] + g_ref[...]) + b_ref[...]
    z = jnp.maximum(z, 0.0)
    h = jnp.dot(z, w_ref[...], preferred_element_type=jnp.float32)
    g2_ref[...] = dinv * h


def _tc_last(cnt_ref, a_ref, g_ref, b_ref, out_ref):
    dinv = _dinv_from_counts(cnt_ref)
    out_ref[...] = dinv * (a_ref[0] + a_ref[1] + g_ref[...]) + b_ref[...]


def _call_tc(body, *args):
    return pl.pallas_call(
        body,
        out_shape=jax.ShapeDtypeStruct((N_PAD, D), jnp.float32),
    )(*args)


def kernel(x, edge_index, W1, b1, W2, b2):
    src = edge_index[0].astype(jnp.int32)
    dst = edge_index[1].astype(jnp.int32)
    n_dummy = E_PAD // NW - E // NW
    dummy = N + (jnp.arange(n_dummy, dtype=jnp.int32) % (N_PAD - N))
    dummy = jnp.broadcast_to(dummy, (NW, n_dummy))
    src3 = jnp.concatenate([src.reshape(NW, E // NW), dummy],
                           axis=1).reshape(NW, CHUNKS, CHUNK)
    dst3 = jnp.concatenate([dst.reshape(NW, E // NW), dummy],
                           axis=1).reshape(NW, CHUNKS, CHUNK)
    x_pad = jnp.pad(x, ((0, N_PAD - N), (0, 0)))

    onesD = jnp.ones((CHUNK, D), jnp.float32)
    zerosD = jnp.zeros((ROWS_SUB, D), jnp.float32)
    b1r = b1.reshape(1, D)
    b2r = b2.reshape(1, D)

    cnt = _sc_degree(dst3, onesD, zerosD)
    g1 = _call_tc(_tc_first, cnt, x_pad, W1)
    a1 = _sc_aggregate(g1, src3, dst3, zerosD)
    g2 = _call_tc(_tc_mid, cnt, a1, g1, b1r, W2)
    a2 = _sc_aggregate(g2, src3, dst3, zerosD)
    out = _call_tc(_tc_last, cnt, a2, g2, b2r)
    return out[:N]


# speedup vs baseline: 28.6396x; 1.0424x over previous
import functools

import jax
import jax.numpy as jnp
from jax import lax
from jax.experimental import pallas as pl
from jax.experimental.pallas import tpu as pltpu
from jax.experimental.pallas import tpu_sc as plsc

N = 10000
E = 320000
D = 128

NC = 2
NS = 16
NW = NC * NS
CHUNK = 125
CHUNKS = 80
E_TILE = CHUNK * CHUNKS
ROWS_SUB = 640
N_PAD = NS * ROWS_SUB
NBUF = 2
H = 2
CH = CHUNKS // H
DEG_RING = 4

_MESH = plsc.VectorSubcoreMesh(core_axis_name="core", subcore_axis_name="subcore")


@functools.partial(
    pl.kernel,
    out_type=jax.ShapeDtypeStruct((NC, N_PAD, D), jnp.float32),
    mesh=_MESH,
    scratch_types=[
        pltpu.VMEM((CHUNKS, CHUNK), jnp.int32),
        pltpu.VMEM((CHUNK, D), jnp.float32),
        pltpu.VMEM_SHARED((N_PAD, D), jnp.float32),
    ] + [pltpu.SemaphoreType.DMA] * DEG_RING,
)
def _sc_degree(edges_hbm, ones_hbm, zeros_hbm, out_hbm, dst_v, ones_v, acc,
               *sems):
    c = lax.axis_index("core")
    s = lax.axis_index("subcore")
    wid = c * NS + s
    pltpu.sync_copy(zeros_hbm, acc.at[pl.ds(s * ROWS_SUB, ROWS_SUB)])
    pltpu.sync_copy(ones_hbm, ones_v)
    pltpu.sync_copy(edges_hbm.at[1, wid], dst_v)
    plsc.subcore_barrier()

    @pl.loop(0, CHUNKS, step=DEG_RING)
    def _(jo):
        for b in range(DEG_RING):
            j = jo + b

            @pl.when(jo > 0)
            def _():
                pltpu.make_async_copy(ones_v, acc.at[dst_v.at[j]],
                                      sems[b]).wait()

            pltpu.async_copy(ones_v, acc.at[dst_v.at[j]], sems[b], add=True)

    for b in range(DEG_RING):
        pltpu.make_async_copy(ones_v, acc.at[dst_v.at[b]], sems[b]).wait()

    plsc.subcore_barrier()
    pltpu.sync_copy(acc.at[pl.ds(s * ROWS_SUB, ROWS_SUB)],
                    out_hbm.at[c, pl.ds(s * ROWS_SUB, ROWS_SUB)])


@functools.partial(
    pl.kernel,
    out_type=jax.ShapeDtypeStruct((NC, N_PAD, D), jnp.float32),
    mesh=_MESH,
    scratch_types=[
        pltpu.VMEM((CH, CHUNK), jnp.int32),
        pltpu.VMEM((CH, CHUNK), jnp.int32),
        pltpu.VMEM((NBUF, CHUNK, D), jnp.float32),
        pltpu.VMEM_SHARED((N_PAD, D), jnp.float32),
    ] + [pltpu.SemaphoreType.DMA] * (2 * NBUF),
)
def _sc_aggregate(g_hbm, edges_hbm, zeros_hbm, out_hbm,
                  src_v, dst_v, rows_v, acc, *sems):
    gsem = sems[:NBUF]
    ssem = sems[NBUF:]
    c = lax.axis_index("core")
    s = lax.axis_index("subcore")
    wid = c * NS + s
    pltpu.sync_copy(zeros_hbm, acc.at[pl.ds(s * ROWS_SUB, ROWS_SUB)])
    plsc.subcore_barrier()

    for h in range(H):
        pltpu.sync_copy(edges_hbm.at[0, wid, pl.ds(h * CH, CH)], src_v)
        pltpu.sync_copy(edges_hbm.at[1, wid, pl.ds(h * CH, CH)], dst_v)
        pltpu.async_copy(g_hbm.at[src_v.at[0]], rows_v.at[0], gsem[0])

        @pl.loop(0, CH, step=NBUF)
        def _(jo):
            for b in range(NBUF):
                j = jo + b
                b1 = 1 - b

                @pl.when(j >= 1)
                def _():
                    pltpu.make_async_copy(rows_v.at[b1], acc.at[dst_v.at[j]],
                                          ssem[b1]).wait()

                jg = jnp.where(j + 1 >= CH, 0, j + 1)
                pltpu.async_copy(g_hbm.at[src_v.at[jg]], rows_v.at[b1],
                                 gsem[b1])
                pltpu.make_async_copy(g_hbm.at[src_v.at[j]], rows_v.at[b],
                                      gsem[b]).wait()
                pltpu.async_copy(rows_v.at[b], acc.at[dst_v.at[j]], ssem[b],
                                 add=True)

        pltpu.make_async_copy(rows_v.at[1], acc.at[dst_v.at[0]],
                              ssem[1]).wait()
        pltpu.make_async_copy(g_hbm.at[src_v.at[0]], rows_v.at[0],
                              gsem[0]).wait()

    plsc.subcore_barrier()
    pltpu.sync_copy(acc.at[pl.ds(s * ROWS_SUB, ROWS_SUB)],
                    out_hbm.at[c, pl.ds(s * ROWS_SUB, ROWS_SUB)])


def _dinv_from_counts(cnt_ref):
    s = (cnt_ref[0] + cnt_ref[1])[:, 0:1].astype(jnp.float32)
    return lax.rsqrt(1.0 + s)


def _tc_first(cnt_ref, x_ref, w_ref, g_ref):
    dinv = _dinv_from_counts(cnt_ref)
    h = jnp.dot(x_ref[...], w_ref[...], preferred_element_type=jnp.float32)
    g_ref[pl.ds(0, N)] = dinv[:N] * h
    g_ref[pl.ds(N, N_PAD - N)] = jnp.zeros((N_PAD - N, D), jnp.float32)


def _tc_mid(cnt_ref, a_ref, g_ref, b_ref, w_ref, g2_ref):
    dinv = _dinv_from_counts(cnt_ref)
    z = dinv * (a_ref[0] + a_ref[1] + g_ref[...]) + b_ref[...]
    z = jnp.maximum(z, 0.0)
    h = jnp.dot(z, w_ref[...], preferred_element_type=jnp.float32)
    g2_ref[...] = dinv * h


def _tc_last(cnt_ref, a_ref, g_ref, b_ref, out_ref):
    dinv = _dinv_from_counts(cnt_ref)
    out_ref[...] = dinv * (a_ref[0] + a_ref[1] + g_ref[...]) + b_ref[...]


def _call_tc(body, *args):
    return pl.pallas_call(
        body,
        out_shape=jax.ShapeDtypeStruct((N_PAD, D), jnp.float32),
    )(*args)


def kernel(x, edge_index, W1, b1, W2, b2):
    edges = edge_index.astype(jnp.int32).reshape(2, NW, CHUNKS, CHUNK)

    onesD = jnp.ones((CHUNK, D), jnp.float32)
    zerosD = jnp.zeros((ROWS_SUB, D), jnp.float32)
    b1r = b1.reshape(1, D)
    b2r = b2.reshape(1, D)

    cnt = _sc_degree(edges, onesD, zerosD)
    g1 = _call_tc(_tc_first, cnt, x, W1)
    a1 = _sc_aggregate(g1, edges, zerosD)
    g2 = _call_tc(_tc_mid, cnt, a1, g1, b1r, W2)
    a2 = _sc_aggregate(g2, edges, zerosD)
    out = _call_tc(_tc_last, cnt, a2, g2, b2r)
    return out[:N]
